# Initial kernel scaffold; baseline (speedup 1.0000x reference)
import jax
import jax.numpy as jnp
from jax.experimental import pallas as pl


def kernel(x, edge_index, edge_attr, fc1_w, fc1_b, k1_w, k1_b, k2_w, k2_b, k3_w, k3_b, root_w, root_b, fc2_w, fc2_b):
    raise NotImplementedError("write your pallas kernel here")


# speedup vs baseline: 3.6973x; 3.6973x over previous
import functools

import jax
import jax.numpy as jnp
from jax import lax
from jax.experimental import pallas as pl
from jax.experimental.pallas import tpu as pltpu
from jax.experimental.pallas import tpu_sc as plsc

_N = 10000
_E = 160000
_W = 16

_NC = 2
_NS = 16
_NW = _NC * _NS
_EPW = _E // _NW
_CH = 40
_NCH = _EPW // _CH
_GRP = 5
_RPT = _N // _NS

_mesh = plsc.VectorSubcoreMesh(core_axis_name="c", subcore_axis_name="s")
_sc_params = pltpu.CompilerParams(use_tc_tiling_on_sc=False)


@functools.partial(
    pl.kernel,
    out_type=jax.ShapeDtypeStruct((_E, _W), jnp.float32),
    mesh=_mesh,
    compiler_params=_sc_params,
    scratch_types=[
        pltpu.VMEM((_NCH, _CH), jnp.int32),
        pltpu.VMEM((_EPW, _W), jnp.float32),
        pltpu.SemaphoreType.DMA,
    ],
)
def _sc_gather(tbl_hbm, idx_hbm, out_hbm, idx_v, rows_v, sem):
    wid = lax.axis_index("s") * _NC + lax.axis_index("c")
    pltpu.sync_copy(idx_hbm.at[wid], idx_v)

    def grp(g, carry):
        for k in range(_GRP):
            j = g * _GRP + k
            pltpu.make_async_copy(
                tbl_hbm.at[idx_v.at[j]], rows_v.at[pl.ds(j * _CH, _CH)], sem
            ).start()
        for k in range(_GRP):
            j = g * _GRP + k
            pltpu.make_async_copy(
                tbl_hbm.at[idx_v.at[j]], rows_v.at[pl.ds(j * _CH, _CH)], sem
            ).wait()
        return carry

    lax.fori_loop(0, _NCH // _GRP, grp, 0)
    pltpu.sync_copy(rows_v, out_hbm.at[pl.ds(wid * _EPW, _EPW)])


@functools.partial(
    pl.kernel,
    out_type=jax.ShapeDtypeStruct((_NC, _N, _W), jnp.float32),
    mesh=_mesh,
    compiler_params=_sc_params,
    scratch_types=[
        pltpu.VMEM((_NCH, _CH), jnp.int32),
        pltpu.VMEM((_EPW, _W), jnp.float32),
        pltpu.VMEM_SHARED((_N, _W), jnp.float32),
    ],
)
def _sc_scatter(msg_hbm, idx_hbm, zeros_hbm, out_hbm, idx_v, rows_v, acc):
    cid = lax.axis_index("c")
    sid = lax.axis_index("s")
    wid = sid * _NC + cid
    pltpu.sync_copy(zeros_hbm.at[pl.ds(sid * _RPT, _RPT)],
                    acc.at[pl.ds(sid * _RPT, _RPT)])
    pltpu.sync_copy(idx_hbm.at[wid], idx_v)
    pltpu.sync_copy(msg_hbm.at[pl.ds(wid * _EPW, _EPW)], rows_v)
    plsc.subcore_barrier()

    def body(j, carry):
        pltpu.sync_copy(rows_v.at[pl.ds(j * _CH, _CH)], acc.at[idx_v.at[j]],
                        add=True)
        return carry

    lax.fori_loop(0, _NCH, body, 0)
    plsc.subcore_barrier()
    pltpu.sync_copy(acc.at[pl.ds(sid * _RPT, _RPT)],
                    out_hbm.at[cid, pl.ds(sid * _RPT, _RPT)])


@functools.partial(
    pl.kernel,
    out_type=jax.ShapeDtypeStruct((_NC, _N, _W), jnp.float32),
    mesh=_mesh,
    compiler_params=_sc_params,
    scratch_types=[
        pltpu.VMEM((_NCH, _CH), jnp.int32),
        pltpu.VMEM((_CH, _W), jnp.float32),
        pltpu.VMEM_SHARED((_N, _W), jnp.float32),
    ],
)
def _sc_deg(idx_hbm, ones_hbm, zeros_hbm, out_hbm, idx_v, ones_v, acc):
    cid = lax.axis_index("c")
    sid = lax.axis_index("s")
    wid = sid * _NC + cid
    pltpu.sync_copy(zeros_hbm.at[pl.ds(sid * _RPT, _RPT)],
                    acc.at[pl.ds(sid * _RPT, _RPT)])
    pltpu.sync_copy(idx_hbm.at[wid], idx_v)
    pltpu.sync_copy(ones_hbm, ones_v)
    plsc.subcore_barrier()

    def body(j, carry):
        pltpu.sync_copy(ones_v, acc.at[idx_v.at[j]], add=True)
        return carry

    lax.fori_loop(0, _NCH, body, 0)
    plsc.subcore_barrier()
    pltpu.sync_copy(acc.at[pl.ds(sid * _RPT, _RPT)],
                    out_hbm.at[cid, pl.ds(sid * _RPT, _RPT)])


def _fc1_body(x_ref, w_ref, b_ref, o_ref):
    o_ref[...] = (
        jnp.dot(x_ref[...], w_ref[...], preferred_element_type=jnp.float32)
        + b_ref[...]
    )


def _tc_fc1(x, w, b):
    return pl.pallas_call(
        _fc1_body,
        out_shape=jax.ShapeDtypeStruct((_N, _W), jnp.float32),
    )(x, w, b)


_BE = 4000


def _edge_body(a_ref, hs_ref, k1, b1, k2, b2, k3, b3, rm, sm, o_ref):
    f32 = jnp.float32
    e1 = jnp.maximum(
        jnp.dot(a_ref[...], k1[...], preferred_element_type=f32) + b1[...], 0.0)
    e2 = jnp.maximum(
        jnp.dot(e1, k2[...], preferred_element_type=f32) + b2[...], 0.0)
    wm = jnp.dot(e2, k3[...], preferred_element_type=f32) + b3[...]
    hrep = jnp.dot(hs_ref[...], rm[...], preferred_element_type=f32)
    o_ref[...] = jnp.dot(wm * hrep, sm[...], preferred_element_type=f32)


def _tc_edge(a, hs, k1, b1, k2, b2, k3, b3, rm, sm):
    def bcast(shape):
        return pl.BlockSpec(shape, lambda i: (0,) * len(shape))

    return pl.pallas_call(
        _edge_body,
        grid=(_E // _BE,),
        in_specs=[
            pl.BlockSpec((_BE, 16), lambda i: (i, 0)),
            pl.BlockSpec((_BE, _W), lambda i: (i, 0)),
            bcast((16, 64)), bcast((1, 64)),
            bcast((64, 128)), bcast((1, 128)),
            bcast((128, 256)), bcast((1, 256)),
            bcast((16, 256)), bcast((256, 16)),
        ],
        out_specs=pl.BlockSpec((_BE, _W), lambda i: (i, 0)),
        out_shape=jax.ShapeDtypeStruct((_E, _W), jnp.float32),
    )(a, hs, k1, b1, k2, b2, k3, b3, rm, sm)


def _update_body(p0, p1, d0, d1, h_ref, rw, rb, o_ref):
    deg = jnp.maximum(d0[:, 0:1] + d1[:, 0:1], 1.0)
    agg = (p0[...] + p1[...]) / deg
    root = jnp.dot(h_ref[...], rw[...], preferred_element_type=jnp.float32)
    o_ref[...] = jnp.maximum(agg + root + rb[...], 0.0)


def _tc_update(p0, p1, d0, d1, h, rw, rb):
    return pl.pallas_call(
        _update_body,
        out_shape=jax.ShapeDtypeStruct((_N, _W), jnp.float32),
    )(p0, p1, d0, d1, h, rw, rb)


def _fc2_body(h_ref, w_ref, b_ref, o_ref):
    o_ref[...] = (
        jnp.dot(h_ref[...], w_ref[...], preferred_element_type=jnp.float32)
        + b_ref[...]
    )


def _tc_fc2(h, w, b):
    return pl.pallas_call(
        _fc2_body,
        out_shape=jax.ShapeDtypeStruct((_N, 1), jnp.float32),
    )(h, w, b)


def kernel(x, edge_index, edge_attr, fc1_w, fc1_b, k1_w, k1_b, k2_w, k2_b,
           k3_w, k3_b, root_w, root_b, fc2_w, fc2_b):
    f32 = jnp.float32
    src = edge_index[0].reshape(_NW, _NCH, _CH)
    dst = edge_index[1].reshape(_NW, _NCH, _CH)
    zeros = jnp.zeros((_N, _W), f32)
    ones_blk = jnp.ones((_CH, _W), f32)
    rmat = jnp.kron(jnp.eye(_W, dtype=f32), jnp.ones((1, _W), f32))
    smat = jnp.kron(jnp.ones((_W, 1), f32), jnp.eye(_W, dtype=f32))

    degp = _sc_deg(dst, ones_blk, zeros)
    h = _tc_fc1(x, fc1_w, fc1_b.reshape(1, _W))
    for k in range(2):
        hs = _sc_gather(h, src)
        msg = _tc_edge(edge_attr, hs,
                       k1_w[k], k1_b[k].reshape(1, -1),
                       k2_w[k], k2_b[k].reshape(1, -1),
                       k3_w[k], k3_b[k].reshape(1, -1),
                       rmat, smat)
        aggp = _sc_scatter(msg, dst, zeros)
        h = _tc_update(aggp[0], aggp[1], degp[0], degp[1], h,
                       root_w[k], root_b[k].reshape(1, _W))
    return _tc_fc2(h, fc2_w, fc2_b.reshape(1, 1))


# speedup vs baseline: 3.8225x; 1.0339x over previous
import functools

import jax
import jax.numpy as jnp
from jax import lax
from jax.experimental import pallas as pl
from jax.experimental.pallas import tpu as pltpu
from jax.experimental.pallas import tpu_sc as plsc

_N = 10000
_E = 160000
_W = 16

_NC = 2
_NS = 16
_NW = _NC * _NS
_EPW = _E // _NW
_CH = 40
_NCH = _EPW // _CH
_GRP = 5
_RPT = _N // _NS

_mesh = plsc.VectorSubcoreMesh(core_axis_name="c", subcore_axis_name="s")
_sc_params = pltpu.CompilerParams(use_tc_tiling_on_sc=False)


@functools.partial(
    pl.kernel,
    out_type=jax.ShapeDtypeStruct((_E, _W), jnp.float32),
    mesh=_mesh,
    compiler_params=_sc_params,
    scratch_types=[
        pltpu.VMEM((_NCH, _CH), jnp.int32),
        pltpu.VMEM((_EPW, _W), jnp.float32),
        pltpu.SemaphoreType.DMA,
    ],
)
def _sc_gather(tbl_hbm, idx_hbm, out_hbm, idx_v, rows_v, sem):
    wid = lax.axis_index("s") * _NC + lax.axis_index("c")
    pltpu.sync_copy(idx_hbm.at[wid], idx_v)

    def grp(g, carry):
        for k in range(_GRP):
            j = g * _GRP + k
            pltpu.make_async_copy(
                tbl_hbm.at[idx_v.at[j]], rows_v.at[pl.ds(j * _CH, _CH)], sem
            ).start()
        for k in range(_GRP):
            j = g * _GRP + k
            pltpu.make_async_copy(
                tbl_hbm.at[idx_v.at[j]], rows_v.at[pl.ds(j * _CH, _CH)], sem
            ).wait()
        return carry

    lax.fori_loop(0, _NCH // _GRP, grp, 0)
    pltpu.sync_copy(rows_v, out_hbm.at[pl.ds(wid * _EPW, _EPW)])


@functools.partial(
    pl.kernel,
    out_type=jax.ShapeDtypeStruct((_NC, _N, _W), jnp.float32),
    mesh=_mesh,
    compiler_params=_sc_params,
    scratch_types=[
        pltpu.VMEM((_NCH, _CH), jnp.int32),
        pltpu.VMEM((_EPW, _W), jnp.float32),
        pltpu.VMEM_SHARED((_N, _W), jnp.float32),
    ],
)
def _sc_scatter(msg_hbm, idx_hbm, zeros_hbm, out_hbm, idx_v, rows_v, acc):
    cid = lax.axis_index("c")
    sid = lax.axis_index("s")
    wid = sid * _NC + cid
    pltpu.sync_copy(zeros_hbm.at[pl.ds(sid * _RPT, _RPT)],
                    acc.at[pl.ds(sid * _RPT, _RPT)])
    pltpu.sync_copy(idx_hbm.at[wid], idx_v)
    pltpu.sync_copy(msg_hbm.at[pl.ds(wid * _EPW, _EPW)], rows_v)
    plsc.subcore_barrier()

    def body(j, carry):
        pltpu.sync_copy(rows_v.at[pl.ds(j * _CH, _CH)], acc.at[idx_v.at[j]],
                        add=True)
        return carry

    lax.fori_loop(0, _NCH, body, 0)
    plsc.subcore_barrier()
    pltpu.sync_copy(acc.at[pl.ds(sid * _RPT, _RPT)],
                    out_hbm.at[cid, pl.ds(sid * _RPT, _RPT)])


@functools.partial(
    pl.kernel,
    out_type=jax.ShapeDtypeStruct((_NC, _N, _W), jnp.float32),
    mesh=_mesh,
    compiler_params=_sc_params,
    scratch_types=[
        pltpu.VMEM((_NCH, _CH), jnp.int32),
        pltpu.VMEM((_CH, _W), jnp.float32),
        pltpu.VMEM_SHARED((_N, _W), jnp.float32),
    ],
)
def _sc_deg(idx_hbm, ones_hbm, zeros_hbm, out_hbm, idx_v, ones_v, acc):
    cid = lax.axis_index("c")
    sid = lax.axis_index("s")
    wid = sid * _NC + cid
    pltpu.sync_copy(zeros_hbm.at[pl.ds(sid * _RPT, _RPT)],
                    acc.at[pl.ds(sid * _RPT, _RPT)])
    pltpu.sync_copy(idx_hbm.at[wid], idx_v)
    pltpu.sync_copy(ones_hbm, ones_v)
    plsc.subcore_barrier()

    def body(j, carry):
        pltpu.sync_copy(ones_v, acc.at[idx_v.at[j]], add=True)
        return carry

    lax.fori_loop(0, _NCH, body, 0)
    plsc.subcore_barrier()
    pltpu.sync_copy(acc.at[pl.ds(sid * _RPT, _RPT)],
                    out_hbm.at[cid, pl.ds(sid * _RPT, _RPT)])


def _fc1_body(x_ref, w_ref, b_ref, o_ref):
    o_ref[...] = (
        jnp.dot(x_ref[...], w_ref[...], preferred_element_type=jnp.float32)
        + b_ref[...]
    )


def _tc_fc1(x, w, b):
    return pl.pallas_call(
        _fc1_body,
        out_shape=jax.ShapeDtypeStruct((_N, _W), jnp.float32),
    )(x, w, b)


_BE = 4000


def _edge_body(a_ref, hs_ref, k1, b1, k2, b2, k3, b3, rm, sm, o_ref):
    f32 = jnp.float32
    bf16 = jnp.bfloat16
    e1 = jnp.maximum(
        jnp.dot(a_ref[...].astype(bf16), k1[...].astype(bf16),
                preferred_element_type=f32) + b1[...], 0.0)
    e2 = jnp.maximum(
        jnp.dot(e1.astype(bf16), k2[...].astype(bf16),
                preferred_element_type=f32) + b2[...], 0.0)
    wm = jnp.dot(e2.astype(bf16), k3[...].astype(bf16),
                 preferred_element_type=f32) + b3[...]
    hrep = jnp.dot(hs_ref[...], rm[...], preferred_element_type=f32)
    o_ref[...] = jnp.dot((wm * hrep).astype(bf16), sm[...].astype(bf16),
                         preferred_element_type=f32)


def _tc_edge(a, hs, k1, b1, k2, b2, k3, b3, rm, sm):
    def bcast(shape):
        return pl.BlockSpec(shape, lambda i: (0,) * len(shape))

    return pl.pallas_call(
        _edge_body,
        grid=(_E // _BE,),
        in_specs=[
            pl.BlockSpec((_BE, 16), lambda i: (i, 0)),
            pl.BlockSpec((_BE, _W), lambda i: (i, 0)),
            bcast((16, 64)), bcast((1, 64)),
            bcast((64, 128)), bcast((1, 128)),
            bcast((128, 256)), bcast((1, 256)),
            bcast((16, 256)), bcast((256, 16)),
        ],
        out_specs=pl.BlockSpec((_BE, _W), lambda i: (i, 0)),
        out_shape=jax.ShapeDtypeStruct((_E, _W), jnp.float32),
    )(a, hs, k1, b1, k2, b2, k3, b3, rm, sm)


def _update_body(p0, p1, d0, d1, h_ref, rw, rb, o_ref):
    deg = jnp.maximum(d0[:, 0:1] + d1[:, 0:1], 1.0)
    agg = (p0[...] + p1[...]) / deg
    root = jnp.dot(h_ref[...], rw[...], preferred_element_type=jnp.float32)
    o_ref[...] = jnp.maximum(agg + root + rb[...], 0.0)


def _tc_update(p0, p1, d0, d1, h, rw, rb):
    return pl.pallas_call(
        _update_body,
        out_shape=jax.ShapeDtypeStruct((_N, _W), jnp.float32),
    )(p0, p1, d0, d1, h, rw, rb)


def _fc2_body(h_ref, w_ref, b_ref, o_ref):
    o_ref[...] = (
        jnp.dot(h_ref[...], w_ref[...], preferred_element_type=jnp.float32)
        + b_ref[...]
    )


def _tc_fc2(h, w, b):
    return pl.pallas_call(
        _fc2_body,
        out_shape=jax.ShapeDtypeStruct((_N, 1), jnp.float32),
    )(h, w, b)


def kernel(x, edge_index, edge_attr, fc1_w, fc1_b, k1_w, k1_b, k2_w, k2_b,
           k3_w, k3_b, root_w, root_b, fc2_w, fc2_b):
    f32 = jnp.float32
    src = edge_index[0].reshape(_NW, _NCH, _CH)
    dst = edge_index[1].reshape(_NW, _NCH, _CH)
    zeros = jnp.zeros((_N, _W), f32)
    ones_blk = jnp.ones((_CH, _W), f32)
    rmat = jnp.kron(jnp.eye(_W, dtype=f32), jnp.ones((1, _W), f32))
    smat = jnp.kron(jnp.ones((_W, 1), f32), jnp.eye(_W, dtype=f32))

    degp = _sc_deg(dst, ones_blk, zeros)
    h = _tc_fc1(x, fc1_w, fc1_b.reshape(1, _W))
    for k in range(2):
        hs = _sc_gather(h, src)
        msg = _tc_edge(edge_attr, hs,
                       k1_w[k], k1_b[k].reshape(1, -1),
                       k2_w[k], k2_b[k].reshape(1, -1),
                       k3_w[k], k3_b[k].reshape(1, -1),
                       rmat, smat)
        aggp = _sc_scatter(msg, dst, zeros)
        h = _tc_update(aggp[0], aggp[1], degp[0], degp[1], h,
                       root_w[k], root_b[k].reshape(1, _W))
    return _tc_fc2(h, fc2_w, fc2_b.reshape(1, 1))


# speedup vs baseline: 5.1838x; 1.3561x over previous
import functools

import jax
import jax.numpy as jnp
from jax import lax
from jax.experimental import pallas as pl
from jax.experimental.pallas import tpu as pltpu
from jax.experimental.pallas import tpu_sc as plsc

_N = 10000
_E = 160000
_W = 16

_NC = 2
_NS = 16
_NW = _NC * _NS
_EPW = _E // _NW
_CH = 40
_NCH = _EPW // _CH
_GRP = 5
_RPT = _N // _NS

_mesh = plsc.VectorSubcoreMesh(core_axis_name="c", subcore_axis_name="s")
_sc_params = pltpu.CompilerParams(use_tc_tiling_on_sc=False)


@functools.partial(
    pl.kernel,
    out_type=jax.ShapeDtypeStruct((_E, _W), jnp.float32),
    mesh=_mesh,
    compiler_params=_sc_params,
    scratch_types=[
        pltpu.VMEM((_NCH, _CH), jnp.int32),
        pltpu.VMEM((_EPW, _W), jnp.float32),
        pltpu.SemaphoreType.DMA,
    ],
)
def _sc_gather(tbl_hbm, idx_hbm, out_hbm, idx_v, rows_v, sem):
    wid = lax.axis_index("s") * _NC + lax.axis_index("c")
    pltpu.sync_copy(idx_hbm.at[wid], idx_v)

    def grp(g, carry):
        for k in range(_GRP):
            j = g * _GRP + k
            pltpu.make_async_copy(
                tbl_hbm.at[idx_v.at[j]], rows_v.at[pl.ds(j * _CH, _CH)], sem
            ).start()
        for k in range(_GRP):
            j = g * _GRP + k
            pltpu.make_async_copy(
                tbl_hbm.at[idx_v.at[j]], rows_v.at[pl.ds(j * _CH, _CH)], sem
            ).wait()
        return carry

    lax.fori_loop(0, _NCH // _GRP, grp, 0)
    pltpu.sync_copy(rows_v, out_hbm.at[pl.ds(wid * _EPW, _EPW)])


@functools.partial(
    pl.kernel,
    out_type=jax.ShapeDtypeStruct((_NC, _N, _W), jnp.float32),
    mesh=_mesh,
    compiler_params=_sc_params,
    scratch_types=[
        pltpu.VMEM((_NCH, _CH), jnp.int32),
        pltpu.VMEM((_EPW, _W), jnp.float32),
        pltpu.VMEM_SHARED((_N, _W), jnp.float32),
    ],
)
def _sc_scatter(msg_hbm, idx_hbm, zeros_hbm, out_hbm, idx_v, rows_v, acc):
    cid = lax.axis_index("c")
    sid = lax.axis_index("s")
    wid = sid * _NC + cid
    pltpu.sync_copy(zeros_hbm.at[pl.ds(sid * _RPT, _RPT)],
                    acc.at[pl.ds(sid * _RPT, _RPT)])
    pltpu.sync_copy(idx_hbm.at[wid], idx_v)
    pltpu.sync_copy(msg_hbm.at[pl.ds(wid * _EPW, _EPW)], rows_v)
    plsc.subcore_barrier()

    def body(j, carry):
        pltpu.sync_copy(rows_v.at[pl.ds(j * _CH, _CH)], acc.at[idx_v.at[j]],
                        add=True)
        return carry

    lax.fori_loop(0, _NCH, body, 0)
    plsc.subcore_barrier()
    pltpu.sync_copy(acc.at[pl.ds(sid * _RPT, _RPT)],
                    out_hbm.at[cid, pl.ds(sid * _RPT, _RPT)])


@functools.partial(
    pl.kernel,
    out_type=jax.ShapeDtypeStruct((_NC, _N, _W), jnp.float32),
    mesh=_mesh,
    compiler_params=_sc_params,
    scratch_types=[
        pltpu.VMEM((_NCH, _CH), jnp.int32),
        pltpu.VMEM((_CH, _W), jnp.float32),
        pltpu.VMEM_SHARED((_N, _W), jnp.float32),
    ],
)
def _sc_deg(idx_hbm, ones_hbm, zeros_hbm, out_hbm, idx_v, ones_v, acc):
    cid = lax.axis_index("c")
    sid = lax.axis_index("s")
    wid = sid * _NC + cid
    pltpu.sync_copy(zeros_hbm.at[pl.ds(sid * _RPT, _RPT)],
                    acc.at[pl.ds(sid * _RPT, _RPT)])
    pltpu.sync_copy(idx_hbm.at[wid], idx_v)
    pltpu.sync_copy(ones_hbm, ones_v)
    plsc.subcore_barrier()

    def body(j, carry):
        pltpu.sync_copy(ones_v, acc.at[idx_v.at[j]], add=True)
        return carry

    lax.fori_loop(0, _NCH, body, 0)
    plsc.subcore_barrier()
    pltpu.sync_copy(acc.at[pl.ds(sid * _RPT, _RPT)],
                    out_hbm.at[cid, pl.ds(sid * _RPT, _RPT)])


def _fc1_body(x_ref, w_ref, b_ref, o_ref):
    o_ref[...] = (
        jnp.dot(x_ref[...], w_ref[...], preferred_element_type=jnp.float32)
        + b_ref[...]
    )


def _tc_fc1(x, w, b):
    return pl.pallas_call(
        _fc1_body,
        out_shape=jax.ShapeDtypeStruct((_N, _W), jnp.float32),
    )(x, w, b)


_BE = 3200
_BR = _BE // 8
_ER = _E // 8


def _unpack(xp):
    return jnp.concatenate([xp[:, 16 * j:16 * (j + 1)] for j in range(8)],
                           axis=0)


def _pack(x):
    return jnp.concatenate([x[_BR * j:_BR * (j + 1), :] for j in range(8)],
                           axis=1)


def _edge_body(a_ref, hs_ref, k1, b1, k2, b2, k3, b3, rm, sm, o_ref):
    f32 = jnp.float32
    bf16 = jnp.bfloat16
    a = _unpack(a_ref[...])
    hs = _unpack(hs_ref[...])
    e1 = jnp.maximum(
        jnp.dot(a.astype(bf16), k1[...].astype(bf16),
                preferred_element_type=f32) + b1[...], 0.0)
    e2 = jnp.maximum(
        jnp.dot(e1.astype(bf16), k2[...].astype(bf16),
                preferred_element_type=f32) + b2[...], 0.0)
    wm = jnp.dot(e2.astype(bf16), k3[...].astype(bf16),
                 preferred_element_type=f32) + b3[...]
    hrep = jnp.dot(hs, rm[...], preferred_element_type=f32)
    msg = jnp.dot((wm * hrep).astype(bf16), sm[...].astype(bf16),
                  preferred_element_type=f32)
    o_ref[...] = _pack(msg)


def _tc_edge(a, hs, k1, b1, k2, b2, k3, b3, rm, sm):
    def bcast(shape):
        return pl.BlockSpec(shape, lambda i: (0,) * len(shape))

    return pl.pallas_call(
        _edge_body,
        grid=(_E // _BE,),
        in_specs=[
            pl.BlockSpec((_BR, 128), lambda i: (i, 0)),
            pl.BlockSpec((_BR, 128), lambda i: (i, 0)),
            bcast((16, 64)), bcast((1, 64)),
            bcast((64, 128)), bcast((1, 128)),
            bcast((128, 256)), bcast((1, 256)),
            bcast((16, 256)), bcast((256, 16)),
        ],
        out_specs=pl.BlockSpec((_BR, 128), lambda i: (i, 0)),
        out_shape=jax.ShapeDtypeStruct((_ER, 128), jnp.float32),
    )(a, hs, k1, b1, k2, b2, k3, b3, rm, sm)


def _update_body(p0, p1, d0, d1, h_ref, rw, rb, o_ref):
    deg = jnp.maximum(d0[:, 0:1] + d1[:, 0:1], 1.0)
    agg = (p0[...] + p1[...]) / deg
    root = jnp.dot(h_ref[...], rw[...], preferred_element_type=jnp.float32)
    o_ref[...] = jnp.maximum(agg + root + rb[...], 0.0)


def _tc_update(p0, p1, d0, d1, h, rw, rb):
    return pl.pallas_call(
        _update_body,
        out_shape=jax.ShapeDtypeStruct((_N, _W), jnp.float32),
    )(p0, p1, d0, d1, h, rw, rb)


def _fc2_body(h_ref, w_ref, b_ref, o_ref):
    o_ref[...] = (
        jnp.dot(h_ref[...], w_ref[...], preferred_element_type=jnp.float32)
        + b_ref[...]
    )


def _tc_fc2(h, w, b):
    return pl.pallas_call(
        _fc2_body,
        out_shape=jax.ShapeDtypeStruct((_N, 1), jnp.float32),
    )(h, w, b)


def kernel(x, edge_index, edge_attr, fc1_w, fc1_b, k1_w, k1_b, k2_w, k2_b,
           k3_w, k3_b, root_w, root_b, fc2_w, fc2_b):
    f32 = jnp.float32
    src = edge_index[0].reshape(_NW, _NCH, _CH)
    dst = edge_index[1].reshape(_NW, _NCH, _CH)
    zeros = jnp.zeros((_N, _W), f32)
    ones_blk = jnp.ones((_CH, _W), f32)
    rmat = jnp.kron(jnp.eye(_W, dtype=f32), jnp.ones((1, _W), f32))
    smat = jnp.kron(jnp.ones((_W, 1), f32), jnp.eye(_W, dtype=f32))

    degp = _sc_deg(dst, ones_blk, zeros)
    h = _tc_fc1(x, fc1_w, fc1_b.reshape(1, _W))
    a128 = edge_attr.reshape(_ER, 128)
    for k in range(2):
        hs = _sc_gather(h, src).reshape(_ER, 128)
        msg = _tc_edge(a128, hs,
                       k1_w[k], k1_b[k].reshape(1, -1),
                       k2_w[k], k2_b[k].reshape(1, -1),
                       k3_w[k], k3_b[k].reshape(1, -1),
                       rmat, smat).reshape(_E, _W)
        aggp = _sc_scatter(msg, dst, zeros)
        h = _tc_update(aggp[0], aggp[1], degp[0], degp[1], h,
                       root_w[k], root_b[k].reshape(1, _W))
    return _tc_fc2(h, fc2_w, fc2_b.reshape(1, 1))


# speedup vs baseline: 5.3888x; 1.0395x over previous
import functools

import jax
import jax.numpy as jnp
from jax import lax
from jax.experimental import pallas as pl
from jax.experimental.pallas import tpu as pltpu
from jax.experimental.pallas import tpu_sc as plsc

_N = 10000
_E = 160000
_W = 16

_NC = 2
_NS = 16
_NW = _NC * _NS
_EPW = _E // _NW
_CH = 40
_NCH = _EPW // _CH
_GRP = 5
_RPT = _N // _NS

_mesh = plsc.VectorSubcoreMesh(core_axis_name="c", subcore_axis_name="s")
_sc_params = pltpu.CompilerParams(use_tc_tiling_on_sc=False)


@functools.partial(
    pl.kernel,
    out_type=jax.ShapeDtypeStruct((_E, _W), jnp.float32),
    mesh=_mesh,
    compiler_params=_sc_params,
    scratch_types=[
        pltpu.VMEM((_NCH, _CH), jnp.int32),
        pltpu.VMEM((_EPW, _W), jnp.float32),
        pltpu.SemaphoreType.DMA,
    ],
)
def _sc_gather(tbl_hbm, idx_hbm, out_hbm, idx_v, rows_v, sem):
    wid = lax.axis_index("s") * _NC + lax.axis_index("c")
    pltpu.sync_copy(idx_hbm.at[wid], idx_v)

    def grp(g, carry):
        for k in range(_GRP):
            j = g * _GRP + k
            pltpu.make_async_copy(
                tbl_hbm.at[idx_v.at[j]], rows_v.at[pl.ds(j * _CH, _CH)], sem
            ).start()
        for k in range(_GRP):
            j = g * _GRP + k
            pltpu.make_async_copy(
                tbl_hbm.at[idx_v.at[j]], rows_v.at[pl.ds(j * _CH, _CH)], sem
            ).wait()
        return carry

    lax.fori_loop(0, _NCH // _GRP, grp, 0)
    pltpu.sync_copy(rows_v, out_hbm.at[pl.ds(wid * _EPW, _EPW)])


@functools.partial(
    pl.kernel,
    out_type=jax.ShapeDtypeStruct((_NC, _N, _W), jnp.float32),
    mesh=_mesh,
    compiler_params=_sc_params,
    scratch_types=[
        pltpu.VMEM((_NCH, _CH), jnp.int32),
        pltpu.VMEM((_EPW, _W), jnp.float32),
        pltpu.VMEM_SHARED((_N, _W), jnp.float32),
        pltpu.SemaphoreType.DMA,
    ],
)
def _sc_scatter(msg_hbm, idx_hbm, zeros_hbm, out_hbm, idx_v, rows_v, acc, sem):
    cid = lax.axis_index("c")
    sid = lax.axis_index("s")
    wid = sid * _NC + cid
    pltpu.sync_copy(zeros_hbm.at[pl.ds(sid * _RPT, _RPT)],
                    acc.at[pl.ds(sid * _RPT, _RPT)])
    pltpu.sync_copy(idx_hbm.at[wid], idx_v)
    pltpu.sync_copy(msg_hbm.at[pl.ds(wid * _EPW, _EPW)], rows_v)
    plsc.subcore_barrier()

    def grp(g, carry):
        for k in range(_GRP):
            j = g * _GRP + k
            pltpu.make_async_copy(
                rows_v.at[pl.ds(j * _CH, _CH)], acc.at[idx_v.at[j]], sem
            ).start(add=True)
        for k in range(_GRP):
            j = g * _GRP + k
            pltpu.make_async_copy(
                rows_v.at[pl.ds(j * _CH, _CH)], acc.at[idx_v.at[j]], sem
            ).wait()
        return carry

    lax.fori_loop(0, _NCH // _GRP, grp, 0)
    plsc.subcore_barrier()
    pltpu.sync_copy(acc.at[pl.ds(sid * _RPT, _RPT)],
                    out_hbm.at[cid, pl.ds(sid * _RPT, _RPT)])


@functools.partial(
    pl.kernel,
    out_type=jax.ShapeDtypeStruct((_NC, _N, _W), jnp.float32),
    mesh=_mesh,
    compiler_params=_sc_params,
    scratch_types=[
        pltpu.VMEM((_NCH, _CH), jnp.int32),
        pltpu.VMEM((_CH, _W), jnp.float32),
        pltpu.VMEM_SHARED((_N, _W), jnp.float32),
        pltpu.SemaphoreType.DMA,
    ],
)
def _sc_deg(idx_hbm, ones_hbm, zeros_hbm, out_hbm, idx_v, ones_v, acc, sem):
    cid = lax.axis_index("c")
    sid = lax.axis_index("s")
    wid = sid * _NC + cid
    pltpu.sync_copy(zeros_hbm.at[pl.ds(sid * _RPT, _RPT)],
                    acc.at[pl.ds(sid * _RPT, _RPT)])
    pltpu.sync_copy(idx_hbm.at[wid], idx_v)
    pltpu.sync_copy(ones_hbm, ones_v)
    plsc.subcore_barrier()

    def grp(g, carry):
        for k in range(_GRP):
            j = g * _GRP + k
            pltpu.make_async_copy(ones_v, acc.at[idx_v.at[j]], sem
                                  ).start(add=True)
        for k in range(_GRP):
            j = g * _GRP + k
            pltpu.make_async_copy(ones_v, acc.at[idx_v.at[j]], sem).wait()
        return carry

    lax.fori_loop(0, _NCH // _GRP, grp, 0)
    plsc.subcore_barrier()
    pltpu.sync_copy(acc.at[pl.ds(sid * _RPT, _RPT)],
                    out_hbm.at[cid, pl.ds(sid * _RPT, _RPT)])


def _fc1_body(x_ref, w_ref, b_ref, o_ref):
    o_ref[...] = (
        jnp.dot(x_ref[...], w_ref[...], preferred_element_type=jnp.float32)
        + b_ref[...]
    )


def _tc_fc1(x, w, b):
    return pl.pallas_call(
        _fc1_body,
        out_shape=jax.ShapeDtypeStruct((_N, _W), jnp.float32),
    )(x, w, b)


_BE = 3200
_BR = _BE // 8
_ER = _E // 8


def _unpack(xp):
    return jnp.concatenate([xp[:, 16 * j:16 * (j + 1)] for j in range(8)],
                           axis=0)


def _pack(x):
    return jnp.concatenate([x[_BR * j:_BR * (j + 1), :] for j in range(8)],
                           axis=1)


def _edge_body(a_ref, hs_ref, k1, b1, k2, b2, k3, b3, rm, sm, o_ref):
    f32 = jnp.float32
    bf16 = jnp.bfloat16
    a = _unpack(a_ref[...])
    hs = _unpack(hs_ref[...])
    e1 = jnp.maximum(
        jnp.dot(a.astype(bf16), k1[...].astype(bf16),
                preferred_element_type=f32) + b1[...], 0.0)
    e2 = jnp.maximum(
        jnp.dot(e1.astype(bf16), k2[...].astype(bf16),
                preferred_element_type=f32) + b2[...], 0.0)
    wm = jnp.dot(e2.astype(bf16), k3[...].astype(bf16),
                 preferred_element_type=f32) + b3[...]
    hrep = jnp.dot(hs, rm[...], preferred_element_type=f32)
    msg = jnp.dot((wm * hrep).astype(bf16), sm[...].astype(bf16),
                  preferred_element_type=f32)
    o_ref[...] = _pack(msg)


def _tc_edge(a, hs, k1, b1, k2, b2, k3, b3, rm, sm):
    def bcast(shape):
        return pl.BlockSpec(shape, lambda i: (0,) * len(shape))

    return pl.pallas_call(
        _edge_body,
        grid=(_E // _BE,),
        in_specs=[
            pl.BlockSpec((_BR, 128), lambda i: (i, 0)),
            pl.BlockSpec((_BR, 128), lambda i: (i, 0)),
            bcast((16, 64)), bcast((1, 64)),
            bcast((64, 128)), bcast((1, 128)),
            bcast((128, 256)), bcast((1, 256)),
            bcast((16, 256)), bcast((256, 16)),
        ],
        out_specs=pl.BlockSpec((_BR, 128), lambda i: (i, 0)),
        out_shape=jax.ShapeDtypeStruct((_ER, 128), jnp.float32),
    )(a, hs, k1, b1, k2, b2, k3, b3, rm, sm)


def _update_body(p0, p1, d0, d1, h_ref, rw, rb, o_ref):
    deg = jnp.maximum(d0[:, 0:1] + d1[:, 0:1], 1.0)
    agg = (p0[...] + p1[...]) / deg
    root = jnp.dot(h_ref[...], rw[...], preferred_element_type=jnp.float32)
    o_ref[...] = jnp.maximum(agg + root + rb[...], 0.0)


def _tc_update(p0, p1, d0, d1, h, rw, rb):
    return pl.pallas_call(
        _update_body,
        out_shape=jax.ShapeDtypeStruct((_N, _W), jnp.float32),
    )(p0, p1, d0, d1, h, rw, rb)


def _updfc2_body(p0, p1, d0, d1, h_ref, rw, rb, w2, b2, o_ref):
    deg = jnp.maximum(d0[:, 0:1] + d1[:, 0:1], 1.0)
    agg = (p0[...] + p1[...]) / deg
    root = jnp.dot(h_ref[...], rw[...], preferred_element_type=jnp.float32)
    h = jnp.maximum(agg + root + rb[...], 0.0)
    o_ref[...] = (
        jnp.dot(h, w2[...], preferred_element_type=jnp.float32) + b2[...]
    )


def _tc_update_fc2(p0, p1, d0, d1, h, rw, rb, w2, b2):
    return pl.pallas_call(
        _updfc2_body,
        out_shape=jax.ShapeDtypeStruct((_N, 1), jnp.float32),
    )(p0, p1, d0, d1, h, rw, rb, w2, b2)


def kernel(x, edge_index, edge_attr, fc1_w, fc1_b, k1_w, k1_b, k2_w, k2_b,
           k3_w, k3_b, root_w, root_b, fc2_w, fc2_b):
    f32 = jnp.float32
    src = edge_index[0].reshape(_NW, _NCH, _CH)
    dst = edge_index[1].reshape(_NW, _NCH, _CH)
    zeros = jnp.zeros((_N, _W), f32)
    ones_blk = jnp.ones((_CH, _W), f32)
    rmat = jnp.kron(jnp.eye(_W, dtype=f32), jnp.ones((1, _W), f32))
    smat = jnp.kron(jnp.ones((_W, 1), f32), jnp.eye(_W, dtype=f32))

    degp = _sc_deg(dst, ones_blk, zeros)
    h = _tc_fc1(x, fc1_w, fc1_b.reshape(1, _W))
    a128 = edge_attr.reshape(_ER, 128)
    for k in range(2):
        hs = _sc_gather(h, src).reshape(_ER, 128)
        msg = _tc_edge(a128, hs,
                       k1_w[k], k1_b[k].reshape(1, -1),
                       k2_w[k], k2_b[k].reshape(1, -1),
                       k3_w[k], k3_b[k].reshape(1, -1),
                       rmat, smat).reshape(_E, _W)
        aggp = _sc_scatter(msg, dst, zeros)
        if k == 0:
            h = _tc_update(aggp[0], aggp[1], degp[0], degp[1], h,
                           root_w[k], root_b[k].reshape(1, _W))
        else:
            return _tc_update_fc2(aggp[0], aggp[1], degp[0], degp[1], h,
                                  root_w[k], root_b[k].reshape(1, _W),
                                  fc2_w, fc2_b.reshape(1, 1))


# speedup vs baseline: 5.5824x; 1.0359x over previous
import functools

import jax
import jax.numpy as jnp
from jax import lax
from jax.experimental import pallas as pl
from jax.experimental.pallas import tpu as pltpu
from jax.experimental.pallas import tpu_sc as plsc

_N = 10000
_E = 160000
_W = 16

_NC = 2
_NS = 16
_NW = _NC * _NS
_EPW = _E // _NW
_CH = 40
_NCH = _EPW // _CH
_GRP = 5
_RPT = _N // _NS

_mesh = plsc.VectorSubcoreMesh(core_axis_name="c", subcore_axis_name="s")
_sc_params = pltpu.CompilerParams(use_tc_tiling_on_sc=False)


@functools.partial(
    pl.kernel,
    out_type=jax.ShapeDtypeStruct((_E, _W), jnp.float32),
    mesh=_mesh,
    compiler_params=_sc_params,
    scratch_types=[
        pltpu.VMEM((_NCH, _CH), jnp.int32),
        pltpu.VMEM((_EPW, _W), jnp.float32),
        pltpu.SemaphoreType.DMA,
    ],
)
def _sc_gather(tbl_hbm, idx_hbm, out_hbm, idx_v, rows_v, sem):
    wid = lax.axis_index("s") * _NC + lax.axis_index("c")
    pltpu.sync_copy(idx_hbm.at[wid], idx_v)

    def grp(g, carry):
        for k in range(_GRP):
            j = g * _GRP + k
            pltpu.make_async_copy(
                tbl_hbm.at[idx_v.at[j]], rows_v.at[pl.ds(j * _CH, _CH)], sem
            ).start()
        for k in range(_GRP):
            j = g * _GRP + k
            pltpu.make_async_copy(
                tbl_hbm.at[idx_v.at[j]], rows_v.at[pl.ds(j * _CH, _CH)], sem
            ).wait()
        return carry

    lax.fori_loop(0, _NCH // _GRP, grp, 0)
    pltpu.sync_copy(rows_v, out_hbm.at[pl.ds(wid * _EPW, _EPW)])


@functools.partial(
    pl.kernel,
    out_type=jax.ShapeDtypeStruct((_NC, _N, _W), jnp.float32),
    mesh=_mesh,
    compiler_params=_sc_params,
    scratch_types=[
        pltpu.VMEM((_NCH, _CH), jnp.int32),
        pltpu.VMEM((_EPW, _W), jnp.float32),
        pltpu.VMEM_SHARED((_N, _W), jnp.float32),
        pltpu.SemaphoreType.DMA,
    ],
)
def _sc_scatter(msg_hbm, idx_hbm, zeros_hbm, out_hbm, idx_v, rows_v, acc, sem):
    cid = lax.axis_index("c")
    sid = lax.axis_index("s")
    wid = sid * _NC + cid
    pltpu.sync_copy(zeros_hbm.at[pl.ds(sid * _RPT, _RPT)],
                    acc.at[pl.ds(sid * _RPT, _RPT)])
    pltpu.sync_copy(idx_hbm.at[wid], idx_v)
    pltpu.sync_copy(msg_hbm.at[pl.ds(wid * _EPW, _EPW)], rows_v)
    plsc.subcore_barrier()

    def grp(g, carry):
        for k in range(_GRP):
            j = g * _GRP + k
            pltpu.make_async_copy(
                rows_v.at[pl.ds(j * _CH, _CH)], acc.at[idx_v.at[j]], sem
            ).start(add=True)
        for k in range(_GRP):
            j = g * _GRP + k
            pltpu.make_async_copy(
                rows_v.at[pl.ds(j * _CH, _CH)], acc.at[idx_v.at[j]], sem
            ).wait()
        return carry

    lax.fori_loop(0, _NCH // _GRP, grp, 0)
    plsc.subcore_barrier()
    pltpu.sync_copy(acc.at[pl.ds(sid * _RPT, _RPT)],
                    out_hbm.at[cid, pl.ds(sid * _RPT, _RPT)])


@functools.partial(
    pl.kernel,
    out_type=(jax.ShapeDtypeStruct((_NC, _N, _W), jnp.float32),
              jax.ShapeDtypeStruct((_NC, _N, _W), jnp.float32)),
    mesh=_mesh,
    compiler_params=_sc_params,
    scratch_types=[
        pltpu.VMEM((_NCH, _CH), jnp.int32),
        pltpu.VMEM((_EPW, _W), jnp.float32),
        pltpu.VMEM((_CH, _W), jnp.float32),
        pltpu.VMEM_SHARED((_N, _W), jnp.float32),
        pltpu.VMEM_SHARED((_N, _W), jnp.float32),
        pltpu.SemaphoreType.DMA,
    ],
)
def _sc_scatter_deg(msg_hbm, idx_hbm, ones_hbm, zeros_hbm, out_hbm, deg_hbm,
                    idx_v, rows_v, ones_v, acc, dacc, sem):
    cid = lax.axis_index("c")
    sid = lax.axis_index("s")
    wid = sid * _NC + cid
    pltpu.sync_copy(zeros_hbm.at[pl.ds(sid * _RPT, _RPT)],
                    acc.at[pl.ds(sid * _RPT, _RPT)])
    pltpu.sync_copy(zeros_hbm.at[pl.ds(sid * _RPT, _RPT)],
                    dacc.at[pl.ds(sid * _RPT, _RPT)])
    pltpu.sync_copy(idx_hbm.at[wid], idx_v)
    pltpu.sync_copy(ones_hbm, ones_v)
    pltpu.sync_copy(msg_hbm.at[pl.ds(wid * _EPW, _EPW)], rows_v)
    plsc.subcore_barrier()

    def grp(g, carry):
        for k in range(_GRP):
            j = g * _GRP + k
            pltpu.make_async_copy(
                rows_v.at[pl.ds(j * _CH, _CH)], acc.at[idx_v.at[j]], sem
            ).start(add=True)
            pltpu.make_async_copy(ones_v, dacc.at[idx_v.at[j]], sem
                                  ).start(add=True)
        for k in range(_GRP):
            j = g * _GRP + k
            pltpu.make_async_copy(
                rows_v.at[pl.ds(j * _CH, _CH)], acc.at[idx_v.at[j]], sem
            ).wait()
            pltpu.make_async_copy(ones_v, dacc.at[idx_v.at[j]], sem).wait()
        return carry

    lax.fori_loop(0, _NCH // _GRP, grp, 0)
    plsc.subcore_barrier()
    pltpu.sync_copy(acc.at[pl.ds(sid * _RPT, _RPT)],
                    out_hbm.at[cid, pl.ds(sid * _RPT, _RPT)])
    pltpu.sync_copy(dacc.at[pl.ds(sid * _RPT, _RPT)],
                    deg_hbm.at[cid, pl.ds(sid * _RPT, _RPT)])


def _fc1_body(x_ref, w_ref, b_ref, o_ref):
    o_ref[...] = (
        jnp.dot(x_ref[...], w_ref[...], preferred_element_type=jnp.float32)
        + b_ref[...]
    )


def _tc_fc1(x, w, b):
    return pl.pallas_call(
        _fc1_body,
        out_shape=jax.ShapeDtypeStruct((_N, _W), jnp.float32),
    )(x, w, b)


_BE = 6400
_BR = _BE // 8
_ER = _E // 8


def _unpack(xp):
    return jnp.concatenate([xp[:, 16 * j:16 * (j + 1)] for j in range(8)],
                           axis=0)


def _pack(x):
    return jnp.concatenate([x[_BR * j:_BR * (j + 1), :] for j in range(8)],
                           axis=1)


def _edge_body(a_ref, hs_ref, k1, b1, k2, b2, k3, b3, rm, sm, o_ref):
    f32 = jnp.float32
    bf16 = jnp.bfloat16
    a = _unpack(a_ref[...])
    hs = _unpack(hs_ref[...])
    e1 = jnp.maximum(
        jnp.dot(a.astype(bf16), k1[...].astype(bf16),
                preferred_element_type=f32) + b1[...], 0.0)
    e2 = jnp.maximum(
        jnp.dot(e1.astype(bf16), k2[...].astype(bf16),
                preferred_element_type=f32) + b2[...], 0.0)
    wm = jnp.dot(e2.astype(bf16), k3[...].astype(bf16),
                 preferred_element_type=f32) + b3[...]
    hrep = jnp.dot(hs.astype(bf16), rm[...].astype(bf16),
                   preferred_element_type=f32)
    msg = jnp.dot((wm * hrep).astype(bf16), sm[...].astype(bf16),
                  preferred_element_type=f32)
    o_ref[...] = _pack(msg)


def _tc_edge(a, hs, k1, b1, k2, b2, k3, b3, rm, sm):
    def bcast(shape):
        return pl.BlockSpec(shape, lambda i: (0,) * len(shape))

    return pl.pallas_call(
        _edge_body,
        grid=(_E // _BE,),
        in_specs=[
            pl.BlockSpec((_BR, 128), lambda i: (i, 0)),
            pl.BlockSpec((_BR, 128), lambda i: (i, 0)),
            bcast((16, 64)), bcast((1, 64)),
            bcast((64, 128)), bcast((1, 128)),
            bcast((128, 256)), bcast((1, 256)),
            bcast((16, 256)), bcast((256, 16)),
        ],
        out_specs=pl.BlockSpec((_BR, 128), lambda i: (i, 0)),
        out_shape=jax.ShapeDtypeStruct((_ER, 128), jnp.float32),
    )(a, hs, k1, b1, k2, b2, k3, b3, rm, sm)


def _update_body(p0, p1, d0, d1, h_ref, rw, rb, o_ref):
    deg = jnp.maximum(d0[:, 0:1] + d1[:, 0:1], 1.0)
    agg = (p0[...] + p1[...]) / deg
    root = jnp.dot(h_ref[...], rw[...], preferred_element_type=jnp.float32)
    o_ref[...] = jnp.maximum(agg + root + rb[...], 0.0)


def _tc_update(p0, p1, d0, d1, h, rw, rb):
    return pl.pallas_call(
        _update_body,
        out_shape=jax.ShapeDtypeStruct((_N, _W), jnp.float32),
    )(p0, p1, d0, d1, h, rw, rb)


def _updfc2_body(p0, p1, d0, d1, h_ref, rw, rb, w2, b2, o_ref):
    deg = jnp.maximum(d0[:, 0:1] + d1[:, 0:1], 1.0)
    agg = (p0[...] + p1[...]) / deg
    root = jnp.dot(h_ref[...], rw[...], preferred_element_type=jnp.float32)
    h = jnp.maximum(agg + root + rb[...], 0.0)
    o_ref[...] = (
        jnp.dot(h, w2[...], preferred_element_type=jnp.float32) + b2[...]
    )


def _tc_update_fc2(p0, p1, d0, d1, h, rw, rb, w2, b2):
    return pl.pallas_call(
        _updfc2_body,
        out_shape=jax.ShapeDtypeStruct((_N, 1), jnp.float32),
    )(p0, p1, d0, d1, h, rw, rb, w2, b2)


def kernel(x, edge_index, edge_attr, fc1_w, fc1_b, k1_w, k1_b, k2_w, k2_b,
           k3_w, k3_b, root_w, root_b, fc2_w, fc2_b):
    f32 = jnp.float32
    src = edge_index[0].reshape(_NW, _NCH, _CH)
    dst = edge_index[1].reshape(_NW, _NCH, _CH)
    zeros = jnp.zeros((_N, _W), f32)
    ones_blk = jnp.ones((_CH, _W), f32)
    rmat = jnp.kron(jnp.eye(_W, dtype=f32), jnp.ones((1, _W), f32))
    smat = jnp.kron(jnp.ones((_W, 1), f32), jnp.eye(_W, dtype=f32))

    h = _tc_fc1(x, fc1_w, fc1_b.reshape(1, _W))
    a128 = edge_attr.reshape(_ER, 128)
    degp = None
    for k in range(2):
        hs = _sc_gather(h, src).reshape(_ER, 128)
        msg = _tc_edge(a128, hs,
                       k1_w[k], k1_b[k].reshape(1, -1),
                       k2_w[k], k2_b[k].reshape(1, -1),
                       k3_w[k], k3_b[k].reshape(1, -1),
                       rmat, smat).reshape(_E, _W)
        if k == 0:
            aggp, degp = _sc_scatter_deg(msg, dst, ones_blk, zeros)
            h = _tc_update(aggp[0], aggp[1], degp[0], degp[1], h,
                           root_w[k], root_b[k].reshape(1, _W))
        else:
            aggp = _sc_scatter(msg, dst, zeros)
            return _tc_update_fc2(aggp[0], aggp[1], degp[0], degp[1], h,
                                  root_w[k], root_b[k].reshape(1, _W),
                                  fc2_w, fc2_b.reshape(1, 1))


# speedup vs baseline: 5.7729x; 1.0341x over previous
import functools

import jax
import jax.numpy as jnp
from jax import lax
from jax.experimental import pallas as pl
from jax.experimental.pallas import tpu as pltpu
from jax.experimental.pallas import tpu_sc as plsc

_N = 10000
_E = 160000
_W = 16

_NC = 2
_NS = 16
_NW = _NC * _NS
_EPW = _E // _NW
_CH = 40
_NCH = _EPW // _CH
_GRP = 5
_RPT = _N // _NS

_mesh = plsc.VectorSubcoreMesh(core_axis_name="c", subcore_axis_name="s")
_sc_params = pltpu.CompilerParams(use_tc_tiling_on_sc=False)


@functools.partial(
    pl.kernel,
    out_type=jax.ShapeDtypeStruct((_E, _W), jnp.float32),
    mesh=_mesh,
    compiler_params=_sc_params,
    scratch_types=[
        pltpu.VMEM((_NCH, _CH), jnp.int32),
        pltpu.VMEM((_EPW, _W), jnp.float32),
        pltpu.SemaphoreType.DMA,
    ],
)
def _sc_gather(tbl_hbm, idx_hbm, out_hbm, idx_v, rows_v, sem):
    wid = lax.axis_index("s") * _NC + lax.axis_index("c")
    pltpu.sync_copy(idx_hbm.at[wid], idx_v)

    def grp(g, carry):
        for k in range(_GRP):
            j = g * _GRP + k
            pltpu.make_async_copy(
                tbl_hbm.at[idx_v.at[j]], rows_v.at[pl.ds(j * _CH, _CH)], sem
            ).start()
        for k in range(_GRP):
            j = g * _GRP + k
            pltpu.make_async_copy(
                tbl_hbm.at[idx_v.at[j]], rows_v.at[pl.ds(j * _CH, _CH)], sem
            ).wait()
        return carry

    lax.fori_loop(0, _NCH // _GRP, grp, 0)
    pltpu.sync_copy(rows_v, out_hbm.at[pl.ds(wid * _EPW, _EPW)])


@functools.partial(
    pl.kernel,
    out_type=jax.ShapeDtypeStruct((_NC, _N, _W), jnp.float32),
    mesh=_mesh,
    compiler_params=_sc_params,
    scratch_types=[
        pltpu.VMEM((_NCH, _CH), jnp.int32),
        pltpu.VMEM((_EPW, _W), jnp.float32),
        pltpu.VMEM_SHARED((_N, _W), jnp.float32),
        pltpu.SemaphoreType.DMA,
    ],
)
def _sc_scatter(msg_hbm, idx_hbm, zeros_hbm, out_hbm, idx_v, rows_v, acc, sem):
    cid = lax.axis_index("c")
    sid = lax.axis_index("s")
    wid = sid * _NC + cid
    pltpu.sync_copy(zeros_hbm.at[pl.ds(sid * _RPT, _RPT)],
                    acc.at[pl.ds(sid * _RPT, _RPT)])
    pltpu.sync_copy(idx_hbm.at[wid], idx_v)
    pltpu.sync_copy(msg_hbm.at[pl.ds(wid * _EPW, _EPW)], rows_v)
    plsc.subcore_barrier()

    def grp(g, carry):
        for k in range(_GRP):
            j = g * _GRP + k
            pltpu.make_async_copy(
                rows_v.at[pl.ds(j * _CH, _CH)], acc.at[idx_v.at[j]], sem
            ).start(add=True)
        for k in range(_GRP):
            j = g * _GRP + k
            pltpu.make_async_copy(
                rows_v.at[pl.ds(j * _CH, _CH)], acc.at[idx_v.at[j]], sem
            ).wait()
        return carry

    lax.fori_loop(0, _NCH // _GRP, grp, 0)
    plsc.subcore_barrier()
    pltpu.sync_copy(acc.at[pl.ds(sid * _RPT, _RPT)],
                    out_hbm.at[cid, pl.ds(sid * _RPT, _RPT)])


@functools.partial(
    pl.kernel,
    out_type=(jax.ShapeDtypeStruct((_NC, _N, _W), jnp.float32),
              jax.ShapeDtypeStruct((_NC, _N, _W), jnp.float32)),
    mesh=_mesh,
    compiler_params=_sc_params,
    scratch_types=[
        pltpu.VMEM((_NCH, _CH), jnp.int32),
        pltpu.VMEM((_EPW, _W), jnp.float32),
        pltpu.VMEM((_CH, _W), jnp.float32),
        pltpu.VMEM_SHARED((_N, _W), jnp.float32),
        pltpu.VMEM_SHARED((_N, _W), jnp.float32),
        pltpu.SemaphoreType.DMA,
    ],
)
def _sc_scatter_deg(msg_hbm, idx_hbm, ones_hbm, zeros_hbm, out_hbm, deg_hbm,
                    idx_v, rows_v, ones_v, acc, dacc, sem):
    cid = lax.axis_index("c")
    sid = lax.axis_index("s")
    wid = sid * _NC + cid
    pltpu.sync_copy(zeros_hbm.at[pl.ds(sid * _RPT, _RPT)],
                    acc.at[pl.ds(sid * _RPT, _RPT)])
    pltpu.sync_copy(zeros_hbm.at[pl.ds(sid * _RPT, _RPT)],
                    dacc.at[pl.ds(sid * _RPT, _RPT)])
    pltpu.sync_copy(idx_hbm.at[wid], idx_v)
    pltpu.sync_copy(ones_hbm, ones_v)
    pltpu.sync_copy(msg_hbm.at[pl.ds(wid * _EPW, _EPW)], rows_v)
    plsc.subcore_barrier()

    def grp(g, carry):
        for k in range(_GRP):
            j = g * _GRP + k
            pltpu.make_async_copy(
                rows_v.at[pl.ds(j * _CH, _CH)], acc.at[idx_v.at[j]], sem
            ).start(add=True)
            pltpu.make_async_copy(ones_v, dacc.at[idx_v.at[j]], sem
                                  ).start(add=True)
        for k in range(_GRP):
            j = g * _GRP + k
            pltpu.make_async_copy(
                rows_v.at[pl.ds(j * _CH, _CH)], acc.at[idx_v.at[j]], sem
            ).wait()
            pltpu.make_async_copy(ones_v, dacc.at[idx_v.at[j]], sem).wait()
        return carry

    lax.fori_loop(0, _NCH // _GRP, grp, 0)
    plsc.subcore_barrier()
    pltpu.sync_copy(acc.at[pl.ds(sid * _RPT, _RPT)],
                    out_hbm.at[cid, pl.ds(sid * _RPT, _RPT)])
    pltpu.sync_copy(dacc.at[pl.ds(sid * _RPT, _RPT)],
                    deg_hbm.at[cid, pl.ds(sid * _RPT, _RPT)])


def _fc1_body(x_ref, w_ref, b_ref, o_ref):
    o_ref[...] = (
        jnp.dot(x_ref[...], w_ref[...], preferred_element_type=jnp.float32)
        + b_ref[...]
    )


def _tc_fc1(x, w, b):
    return pl.pallas_call(
        _fc1_body,
        out_shape=jax.ShapeDtypeStruct((_N, _W), jnp.float32),
    )(x, w, b)


_BE = 6400
_BR = _BE // 8
_ER = _E // 8


def _unpack(xp):
    return jnp.concatenate([xp[:, 16 * j:16 * (j + 1)] for j in range(8)],
                           axis=0)


def _pack(x):
    return jnp.concatenate([x[_BR * j:_BR * (j + 1), :] for j in range(8)],
                           axis=1)


def _edge_body(a_ref, hs_ref, k1, b1, k2, b2, k3, b3, rm, sm, o_ref):
    f32 = jnp.float32
    bf16 = jnp.bfloat16
    hs = _unpack(hs_ref[...])
    e1 = jnp.maximum(
        lax.dot_general(a_ref[...].astype(bf16), k1[...].astype(bf16),
                        (((0,), (0,)), ((), ())),
                        preferred_element_type=f32) + b1[...], 0.0)
    e2 = jnp.maximum(
        jnp.dot(e1.astype(bf16), k2[...].astype(bf16),
                preferred_element_type=f32) + b2[...], 0.0)
    wm = jnp.dot(e2.astype(bf16), k3[...].astype(bf16),
                 preferred_element_type=f32) + b3[...]
    hrep = jnp.dot(hs.astype(bf16), rm[...].astype(bf16),
                   preferred_element_type=f32)
    msg = jnp.dot((wm * hrep).astype(bf16), sm[...].astype(bf16),
                  preferred_element_type=f32)
    o_ref[...] = _pack(msg)


def _tc_edge(a, hs, k1, b1, k2, b2, k3, b3, rm, sm):
    def bcast(shape):
        return pl.BlockSpec(shape, lambda i: (0,) * len(shape))

    return pl.pallas_call(
        _edge_body,
        grid=(_E // _BE,),
        in_specs=[
            pl.BlockSpec((16, _BE), lambda i: (0, i)),
            pl.BlockSpec((_BR, 128), lambda i: (i, 0)),
            bcast((16, 64)), bcast((1, 64)),
            bcast((64, 128)), bcast((1, 128)),
            bcast((128, 256)), bcast((1, 256)),
            bcast((16, 256)), bcast((256, 16)),
        ],
        out_specs=pl.BlockSpec((_BR, 128), lambda i: (i, 0)),
        out_shape=jax.ShapeDtypeStruct((_ER, 128), jnp.float32),
    )(a, hs, k1, b1, k2, b2, k3, b3, rm, sm)


def _update_body(p0, p1, d0, d1, h_ref, rw, rb, o_ref):
    deg = jnp.maximum(d0[:, 0:1] + d1[:, 0:1], 1.0)
    agg = (p0[...] + p1[...]) / deg
    root = jnp.dot(h_ref[...], rw[...], preferred_element_type=jnp.float32)
    o_ref[...] = jnp.maximum(agg + root + rb[...], 0.0)


def _tc_update(p0, p1, d0, d1, h, rw, rb):
    return pl.pallas_call(
        _update_body,
        out_shape=jax.ShapeDtypeStruct((_N, _W), jnp.float32),
    )(p0, p1, d0, d1, h, rw, rb)


def _updfc2_body(p0, p1, d0, d1, h_ref, rw, rb, w2, b2, o_ref):
    deg = jnp.maximum(d0[:, 0:1] + d1[:, 0:1], 1.0)
    agg = (p0[...] + p1[...]) / deg
    root = jnp.dot(h_ref[...], rw[...], preferred_element_type=jnp.float32)
    h = jnp.maximum(agg + root + rb[...], 0.0)
    o_ref[...] = (
        jnp.dot(h, w2[...], preferred_element_type=jnp.float32) + b2[...]
    )


def _tc_update_fc2(p0, p1, d0, d1, h, rw, rb, w2, b2):
    return pl.pallas_call(
        _updfc2_body,
        out_shape=jax.ShapeDtypeStruct((_N, 1), jnp.float32),
    )(p0, p1, d0, d1, h, rw, rb, w2, b2)


def kernel(x, edge_index, edge_attr, fc1_w, fc1_b, k1_w, k1_b, k2_w, k2_b,
           k3_w, k3_b, root_w, root_b, fc2_w, fc2_b):
    f32 = jnp.float32

    def slotify(v):
        return (v.reshape(_E // _BE, 8, _BR).transpose(0, 2, 1)
                .reshape(_NW, _NCH, _CH))

    src = slotify(edge_index[0])
    dst = slotify(edge_index[1])
    zeros = jnp.zeros((_N, _W), f32)
    ones_blk = jnp.ones((_CH, _W), f32)
    rmat = jnp.kron(jnp.eye(_W, dtype=f32), jnp.ones((1, _W), f32))
    smat = jnp.kron(jnp.ones((_W, 1), f32), jnp.eye(_W, dtype=f32))

    h = _tc_fc1(x, fc1_w, fc1_b.reshape(1, _W))
    a_t = edge_attr.T
    degp = None
    for k in range(2):
        hs = _sc_gather(h, src).reshape(_ER, 128)
        msg = _tc_edge(a_t, hs,
                       k1_w[k], k1_b[k].reshape(1, -1),
                       k2_w[k], k2_b[k].reshape(1, -1),
                       k3_w[k], k3_b[k].reshape(1, -1),
                       rmat, smat).reshape(_E, _W)
        if k == 0:
            aggp, degp = _sc_scatter_deg(msg, dst, ones_blk, zeros)
            h = _tc_update(aggp[0], aggp[1], degp[0], degp[1], h,
                           root_w[k], root_b[k].reshape(1, _W))
        else:
            aggp = _sc_scatter(msg, dst, zeros)
            return _tc_update_fc2(aggp[0], aggp[1], degp[0], degp[1], h,
                                  root_w[k], root_b[k].reshape(1, _W),
                                  fc2_w, fc2_b.reshape(1, 1))
